# baseline (speedup 1.0000x reference)
import jax
import jax.numpy as jnp
from jax import lax
from jax.experimental import pallas as pl
from jax.experimental.pallas import tpu as pltpu
from jax.experimental.pallas import tpu_sc as plsc

D = 64
B = 4096
VP = 50000

NC = 2
NS = 16
L = 16
NW = NC * NS
BPW = B // NW
CH = D // L
NSEM = 8


def _gmf_body(uid_hbm, iid_hbm, ut_hbm, it_hbm, w_hbm, bb_hbm, out_hbm,
              uid_v, iid_v, u_rows, v_rows,
              w_v, bb_v, o_v, *sems):
    wid = lax.axis_index("s") * NC + lax.axis_index("c")
    base = wid * BPW

    pltpu.sync_copy(uid_hbm.at[pl.ds(base, BPW)], uid_v)
    pltpu.sync_copy(iid_hbm.at[pl.ds(base, BPW)], iid_v)
    cps = []
    for g in range(BPW // L):
        uof = uid_v[pl.ds(g * L, L)] * D
        iof = iid_v[pl.ds(g * L, L)] * D
        for r in range(L):
            j = g * L + r
            uo = pl.multiple_of(uof[r], D)
            io = pl.multiple_of(iof[r], D)
            cps.append(pltpu.async_copy(
                ut_hbm.at[pl.ds(uo, D)], u_rows.at[j], sems[(2 * j) % NSEM]))
            cps.append(pltpu.async_copy(
                it_hbm.at[pl.ds(io, D)], v_rows.at[j], sems[(2 * j + 1) % NSEM]))
    pltpu.sync_copy(w_hbm, w_v)
    pltpu.sync_copy(bb_hbm, bb_v)

    wc = [w_v[pl.ds(L * c, L)] for c in range(CH)]
    bias = bb_v[...]
    lane = lax.iota(jnp.int32, L)
    masks = [lane == r for r in range(L)]

    for cp in cps:
        cp.wait()

    for g in range(BPW // L):
        acc = jnp.zeros((L,), jnp.float32)
        for r in range(L):
            j = g * L + r
            part = (u_rows[j, pl.ds(0, L)]
                    * v_rows[j, pl.ds(0, L)] * wc[0])
            for c in range(1, CH):
                part = part + (u_rows[j, pl.ds(L * c, L)]
                               * v_rows[j, pl.ds(L * c, L)] * wc[c])
            acc = jnp.where(masks[r], jnp.sum(part), acc)
        x = acc + bias
        o_v[pl.ds(g * L, L)] = 1.0 / (1.0 + jnp.exp(-x))

    pltpu.sync_copy(o_v, out_hbm.at[pl.ds(base, BPW)])


@jax.jit
def _gmf_sc(user_ids, item_ids, user_pairs, item_pairs, w_flat, b_vec):
    mesh = plsc.VectorSubcoreMesh(
        core_axis_name="c", subcore_axis_name="s",
        num_cores=NC, num_subcores=NS)
    run = pl.kernel(
        _gmf_body,
        mesh=mesh,
        compiler_params=pltpu.CompilerParams(needs_layout_passes=False),
        out_type=jax.ShapeDtypeStruct((B,), jnp.float32),
        scratch_types=[
            pltpu.VMEM((BPW,), jnp.int32),
            pltpu.VMEM((BPW,), jnp.int32),
            pltpu.VMEM((BPW, D), jnp.float32),
            pltpu.VMEM((BPW, D), jnp.float32),
            pltpu.VMEM((D,), jnp.float32),
            pltpu.VMEM((L,), jnp.float32),
            pltpu.VMEM((BPW,), jnp.float32),
        ] + [pltpu.SemaphoreType.DMA] * NSEM + [
        ],
    )
    return run(user_ids, item_ids, user_pairs, item_pairs, w_flat, b_vec)


def kernel(user_ids, item_ids, user_table, item_table, W, b):
    user_pairs = user_table.reshape(VP * 2 * D)
    item_pairs = item_table.reshape(VP * 2 * D)
    w_flat = W.reshape(D)
    b_vec = jnp.broadcast_to(b.astype(jnp.float32), (L,))
    out = _gmf_sc(user_ids, item_ids, user_pairs, item_pairs,
                  w_flat, b_vec)
    return out.reshape(B, 1)


# speedup vs baseline: 1.3351x; 1.3351x over previous
import jax
import jax.numpy as jnp
from jax import lax
from jax.experimental import pallas as pl
from jax.experimental.pallas import tpu as pltpu
from jax.experimental.pallas import tpu_sc as plsc

D = 64
B = 4096

NC = 2
NS = 16
L = 16
NW = NC * NS
BPW = B // NW
CH = D // L
TR = 8
HALF = BPW // 2
NSEM = 8


def _gather_scale_body(ids_hbm, tab_hbm, w_hbm, out_hbm,
                       ids_v, blk_v, rows, w_v, o_v, *sems):
    wid = lax.axis_index("s") * NC + lax.axis_index("c")
    base = wid * BPW

    pltpu.sync_copy(ids_hbm.at[pl.ds(base, BPW)], ids_v)
    for k in range(BPW // L):
        sl = pl.ds(k * L, L)
        blk_v[sl] = ids_v[sl] & jnp.int32(-TR)
    pltpu.sync_copy(w_hbm, w_v)
    wc = [w_v[pl.ds(L * c, L)] for c in range(CH)]

    for h in range(BPW // HALF):
        cps = []
        for g in range(HALF // L):
            bch = blk_v[pl.ds(h * HALF + g * L, L)]
            for r in range(L):
                j = g * L + r
                bo = pl.multiple_of(bch[r], TR)
                cps.append(pltpu.async_copy(
                    tab_hbm.at[pl.ds(bo, TR)], rows.at[j], sems[j % NSEM]))
        for cp in cps:
            cp.wait()
        for g in range(HALF // L):
            sub = ids_v[pl.ds(h * HALF + g * L, L)] & 7
            for r in range(L):
                j = g * L + r
                ro = sub[r]
                for c in range(CH):
                    o_v[h * HALF + g * L + r, pl.ds(L * c, L)] = (
                        rows[j, ro, pl.ds(L * c, L)] * wc[c])

    pltpu.sync_copy(o_v, out_hbm.at[pl.ds(base, BPW)])


def _gather_combine_body(ids_hbm, tab_hbm, uw_hbm, bb_hbm, out_hbm,
                         ids_v, blk_v, rows, uw_v, bb_v, o_v, *sems):
    wid = lax.axis_index("s") * NC + lax.axis_index("c")
    base = wid * BPW

    pltpu.sync_copy(ids_hbm.at[pl.ds(base, BPW)], ids_v)
    for k in range(BPW // L):
        sl = pl.ds(k * L, L)
        blk_v[sl] = ids_v[sl] & jnp.int32(-TR)
    pltpu.sync_copy(uw_hbm.at[pl.ds(base, BPW)], uw_v)
    pltpu.sync_copy(bb_hbm, bb_v)
    bias = bb_v[...]
    lane = lax.iota(jnp.int32, L)
    masks = [lane == r for r in range(L)]

    for h in range(BPW // HALF):
        cps = []
        for g in range(HALF // L):
            bch = blk_v[pl.ds(h * HALF + g * L, L)]
            for r in range(L):
                j = g * L + r
                bo = pl.multiple_of(bch[r], TR)
                cps.append(pltpu.async_copy(
                    tab_hbm.at[pl.ds(bo, TR)], rows.at[j], sems[j % NSEM]))
        for cp in cps:
            cp.wait()
        for g in range(HALF // L):
            acc = jnp.zeros((L,), jnp.float32)
            sub = ids_v[pl.ds(h * HALF + g * L, L)] & 7
            for r in range(L):
                j = g * L + r
                ro = sub[r]
                jj = h * HALF + g * L + r
                part = rows[j, ro, pl.ds(0, L)] * uw_v[jj, pl.ds(0, L)]
                for c in range(1, CH):
                    part = part + (rows[j, ro, pl.ds(L * c, L)]
                                   * uw_v[jj, pl.ds(L * c, L)])
                acc = jnp.where(masks[r], jnp.sum(part), acc)
            x = acc + bias
            o_v[pl.ds(h * HALF + g * L, L)] = 1.0 / (1.0 + jnp.exp(-x))

    pltpu.sync_copy(o_v, out_hbm.at[pl.ds(base, BPW)])


@jax.jit
def _gmf_sc2(user_ids, item_ids, user_table, item_table, w_flat, b_vec):
    mesh = plsc.VectorSubcoreMesh(
        core_axis_name="c", subcore_axis_name="s",
        num_cores=NC, num_subcores=NS)
    uw = pl.kernel(
        _gather_scale_body,
        mesh=mesh,
        compiler_params=pltpu.CompilerParams(needs_layout_passes=False),
        out_type=jax.ShapeDtypeStruct((B, D), jnp.float32),
        scratch_types=[
            pltpu.VMEM((BPW,), jnp.int32),
            pltpu.VMEM((BPW,), jnp.int32),
            pltpu.VMEM((HALF, TR, D), jnp.float32),
            pltpu.VMEM((D,), jnp.float32),
            pltpu.VMEM((BPW, D), jnp.float32),
        ] + [pltpu.SemaphoreType.DMA] * NSEM,
    )(user_ids, user_table, w_flat)
    out = pl.kernel(
        _gather_combine_body,
        mesh=mesh,
        compiler_params=pltpu.CompilerParams(needs_layout_passes=False),
        out_type=jax.ShapeDtypeStruct((B,), jnp.float32),
        scratch_types=[
            pltpu.VMEM((BPW,), jnp.int32),
            pltpu.VMEM((BPW,), jnp.int32),
            pltpu.VMEM((HALF, TR, D), jnp.float32),
            pltpu.VMEM((BPW, D), jnp.float32),
            pltpu.VMEM((L,), jnp.float32),
            pltpu.VMEM((BPW,), jnp.float32),
        ] + [pltpu.SemaphoreType.DMA] * NSEM,
    )(item_ids, item_table, uw, b_vec)
    return out


def kernel(user_ids, item_ids, user_table, item_table, W, b):
    w_flat = W.reshape(D)
    b_vec = jnp.broadcast_to(b.astype(jnp.float32), (L,))
    out = _gmf_sc2(user_ids, item_ids, user_table, item_table,
                   w_flat, b_vec)
    return out.reshape(B, 1)
